# baseline (speedup 1.0000x reference)
import functools

import jax
import jax.numpy as jnp
from jax import lax
from jax.experimental import pallas as pl
from jax.experimental.pallas import tpu as pltpu
from jax.experimental.pallas import tpu_sc as plsc

EMBED = 32
LANES = 16
CHUNK = 64
PROJ_BLK = 65536


NBUF = 4


def _proj_stream_body(nblk, w_ref, tab_ref, out_ref, buf, obuf, sem_in, sem_out):
    def in_copy(k, b):
        return pltpu.make_async_copy(
            tab_ref.at[:, pl.ds(k * PROJ_BLK, PROJ_BLK)],
            buf.at[b],
            sem_in.at[b])

    def out_copy(k, b):
        return pltpu.make_async_copy(
            obuf.at[b],
            out_ref.at[pl.ds(k * PROJ_BLK, PROJ_BLK)],
            sem_out.at[b])

    for k in range(min(NBUF, nblk)):
        in_copy(k, k).start()
    for k in range(nblk):
        b = k % NBUF
        in_copy(k, b).wait()
        o = jnp.dot(w_ref[...], buf[b],
                    preferred_element_type=jnp.float32)[0]
        if k >= NBUF:
            out_copy(k - NBUF, b).wait()
        obuf[b, :] = o
        out_copy(k, b).start()
        if k + NBUF < nblk:
            in_copy(k + NBUF, b).start()
    for k in range(max(nblk - NBUF, 0), nblk):
        out_copy(k, k % NBUF).wait()


def _proj_tail_body(w_ref, tab_ref, init_ref, out_ref):
    del init_ref
    out_ref[...] = jnp.dot(
        w_ref[...], tab_ref[...], preferred_element_type=jnp.float32)[0]


def _tc_project(w_row, tab_t):
    n = tab_t.shape[1]
    nblk = n // PROJ_BLK
    main = pl.pallas_call(
        functools.partial(_proj_stream_body, nblk),
        in_specs=[
            pl.BlockSpec((1, EMBED), lambda: (0, 0)),
            pl.BlockSpec(memory_space=pl.ANY),
        ],
        out_specs=pl.BlockSpec(memory_space=pl.ANY),
        out_shape=jax.ShapeDtypeStruct((n,), jnp.float32),
        scratch_shapes=[
            pltpu.VMEM((NBUF, EMBED, PROJ_BLK), jnp.float32),
            pltpu.VMEM((NBUF, PROJ_BLK), jnp.float32),
            pltpu.SemaphoreType.DMA((NBUF,)),
            pltpu.SemaphoreType.DMA((NBUF,)),
        ],
    )(w_row, tab_t)
    if n % PROJ_BLK == 0:
        return main
    return pl.pallas_call(
        _proj_tail_body,
        grid=(1,),
        in_specs=[
            pl.BlockSpec((1, EMBED), lambda i: (0, 0)),
            pl.BlockSpec((EMBED, PROJ_BLK), lambda i: (0, nblk)),
            pl.BlockSpec(memory_space=pl.ANY),
        ],
        out_specs=pl.BlockSpec((PROJ_BLK,), lambda i: (nblk,)),
        out_shape=jax.ShapeDtypeStruct((n,), jnp.float32),
        input_output_aliases={2: 0},
    )(w_row, tab_t, main)


def _sc_body(nc, bpw, idx_h, proj_h, bv_h, out_h,
             idx_v, g_v, bv_v, out_v, sem_i, sem_b, sem):
    wid = lax.axis_index("s") * nc + lax.axis_index("c")
    base = wid * bpw
    nch = bpw // CHUNK
    elementwise = bv_h.shape[0] != LANES

    idx_cp = pltpu.async_copy(idx_h.at[pl.ds(base, bpw)], idx_v, sem_i)
    if elementwise:
        bv_cp = pltpu.async_copy(bv_h.at[pl.ds(base, bpw)], bv_v, sem_b)
    else:
        bv_cp = pltpu.async_copy(bv_h, bv_v, sem_b)

    idx_cp.wait()
    copies = []
    for j in range(nch):
        copies.append(pltpu.async_copy(
            proj_h.at[idx_v.at[pl.ds(j * CHUNK, CHUNK)]],
            g_v.at[pl.ds(j * CHUNK, CHUNK)], sem))
    bv_cp.wait()
    for c in copies:
        c.wait()

    for i in range(0, bpw, LANES):
        bval = bv_v[pl.ds(i, LANES)] if elementwise else bv_v[...]
        out_v[pl.ds(i, LANES)] = g_v[pl.ds(i, LANES)] + bval

    pltpu.sync_copy(out_v, out_h.at[pl.ds(base, bpw)])


def _sc_gather_add(idx, proj, base_vals):
    batch = idx.shape[0]
    info = plsc.get_sparse_core_info()
    nc, ns = info.num_cores, info.num_subcores
    bpw = batch // (nc * ns)

    bv_shape = (LANES,) if base_vals.shape[0] == LANES else (bpw,)
    mesh = plsc.VectorSubcoreMesh(core_axis_name="c", subcore_axis_name="s")
    fn = pl.kernel(
        functools.partial(_sc_body, nc, bpw),
        out_type=jax.ShapeDtypeStruct((batch,), jnp.float32),
        mesh=mesh,
        compiler_params=pltpu.CompilerParams(
            needs_layout_passes=False, use_tc_tiling_on_sc=False),
        scratch_types=[
            pltpu.VMEM((bpw,), jnp.int32),
            pltpu.VMEM((bpw,), jnp.float32),
            pltpu.VMEM(bv_shape, jnp.float32),
            pltpu.VMEM((bpw,), jnp.float32),
            pltpu.SemaphoreType.DMA,
            pltpu.SemaphoreType.DMA,
            pltpu.SemaphoreType.DMA,
        ],
    )
    return fn(idx, proj, base_vals)


@jax.jit
def _run(user, course, user_table, course_table, W, b):
    w_u = W[:, :EMBED]
    w_c = W[:, EMBED:]
    b_vec = jnp.broadcast_to(b, (LANES,)).astype(jnp.float32)
    proj_c = _tc_project(w_c, course_table.T)
    partial = _sc_gather_add(course, proj_c, b_vec)
    proj_u = _tc_project(w_u, user_table.T)
    return _sc_gather_add(user, proj_u, partial)


def kernel(user, course, user_table, course_table, W, b):
    out = _run(user, course, user_table, course_table, W, b)
    return out.reshape(-1, 1)


# speedup vs baseline: 1.1105x; 1.1105x over previous
import functools

import jax
import jax.numpy as jnp
from jax import lax
from jax.experimental import pallas as pl
from jax.experimental.pallas import tpu as pltpu
from jax.experimental.pallas import tpu_sc as plsc

EMBED = 32
LANES = 16
CHUNK = 64
PROJ_BLK = 65536


def _proj_body(w_ref, tab_ref, out_ref):
    out_ref[...] = jnp.dot(
        w_ref[...], tab_ref[...], preferred_element_type=jnp.float32)[0]


def _tc_project(w_row, tab_t):
    n = tab_t.shape[1]
    grid = pl.cdiv(n, PROJ_BLK)
    return pl.pallas_call(
        _proj_body,
        grid=(grid,),
        in_specs=[
            pl.BlockSpec((1, EMBED), lambda i: (0, 0)),
            pl.BlockSpec((EMBED, PROJ_BLK), lambda i: (0, i)),
        ],
        out_specs=pl.BlockSpec((PROJ_BLK,), lambda i: (i,)),
        out_shape=jax.ShapeDtypeStruct((n,), jnp.float32),
    )(w_row, tab_t)


def _to_hbm(x):
    return pltpu.with_memory_space_constraint(x, pltpu.MemorySpace.HBM)


def _sc_body(nc, bpw, idx_h, proj_h, bv_h, out_h,
             idx_v, g_v, bv_v, out_v, sem_i, sem_b, sem):
    wid = lax.axis_index("s") * nc + lax.axis_index("c")
    base = wid * bpw
    nch = bpw // CHUNK
    elementwise = bv_h.shape[0] != LANES

    idx_cp = pltpu.async_copy(idx_h.at[pl.ds(base, bpw)], idx_v, sem_i)
    if elementwise:
        bv_cp = pltpu.async_copy(bv_h.at[pl.ds(base, bpw)], bv_v, sem_b)
    else:
        bv_cp = pltpu.async_copy(bv_h, bv_v, sem_b)

    idx_cp.wait()
    copies = []
    for j in range(nch):
        copies.append(pltpu.async_copy(
            proj_h.at[idx_v.at[pl.ds(j * CHUNK, CHUNK)]],
            g_v.at[pl.ds(j * CHUNK, CHUNK)], sem))
    bv_cp.wait()
    for c in copies:
        c.wait()

    for i in range(0, bpw, LANES):
        bval = bv_v[pl.ds(i, LANES)] if elementwise else bv_v[...]
        out_v[pl.ds(i, LANES)] = g_v[pl.ds(i, LANES)] + bval

    pltpu.sync_copy(out_v, out_h.at[pl.ds(base, bpw)])


def _sc_gather_add(idx, proj, base_vals):
    batch = idx.shape[0]
    info = plsc.get_sparse_core_info()
    nc, ns = info.num_cores, info.num_subcores
    bpw = batch // (nc * ns)

    bv_shape = (LANES,) if base_vals.shape[0] == LANES else (bpw,)
    mesh = plsc.VectorSubcoreMesh(core_axis_name="c", subcore_axis_name="s")
    fn = pl.kernel(
        functools.partial(_sc_body, nc, bpw),
        out_type=jax.ShapeDtypeStruct((batch,), jnp.float32),
        mesh=mesh,
        compiler_params=pltpu.CompilerParams(
            needs_layout_passes=False, use_tc_tiling_on_sc=False),
        scratch_types=[
            pltpu.VMEM((bpw,), jnp.int32),
            pltpu.VMEM((bpw,), jnp.float32),
            pltpu.VMEM(bv_shape, jnp.float32),
            pltpu.VMEM((bpw,), jnp.float32),
            pltpu.SemaphoreType.DMA,
            pltpu.SemaphoreType.DMA,
            pltpu.SemaphoreType.DMA,
        ],
    )
    return fn(idx, proj, base_vals)


@jax.jit
def _run(user, course, user_table, course_table, W, b):
    w_u = W[:, :EMBED]
    w_c = W[:, EMBED:]
    b_vec = jnp.broadcast_to(b, (LANES,)).astype(jnp.float32)
    proj_c = _to_hbm(_tc_project(w_c, course_table.T))
    partial = _sc_gather_add(course, proj_c, b_vec)
    proj_u = _to_hbm(_tc_project(w_u, user_table.T))
    return _sc_gather_add(user, proj_u, partial)


def kernel(user, course, user_table, course_table, W, b):
    out = _run(user, course, user_table, course_table, W, b)
    return out.reshape(-1, 1)
